# baseline (speedup 1.0000x reference)
import jax
import jax.numpy as jnp
from jax import lax
from jax.experimental import pallas as pl
from jax.experimental.pallas import tpu as pltpu
from jax.experimental.pallas import tpu_sc as plsc

N = 100000
E = 1600000
G = 128
FH = 16

NC, NS = 2, 16
SW = 128
E_PAD = 1638400
ROWS_2D = E_PAD // SW
SRC_PAD = N
DST_PAD = N

A_CHUNK_ROWS = 16
A_CHUNKS = 25
A_TILE_ROWS = A_CHUNK_ROWS * A_CHUNKS
DEG_PAD = 100352
DEG_SLICE = DEG_PAD // NS

C_CHUNK_ROWS = 4
C_CHUNKS = 200
C_TILE_ROWS = C_CHUNK_ROWS * C_CHUNKS
C_BLK = C_CHUNK_ROWS * SW
AGG_PAD = 100480
DUMMY = N

R_BLK = 2000
N_BLKS = N // R_BLK


def _deg_body(src2d, zeros_hbm, deg_out, deg_sp, idx_v, ones_v, sem):
    c = lax.axis_index("c")
    s = lax.axis_index("s")
    pltpu.sync_copy(zeros_hbm.at[pl.ds(s * DEG_SLICE, DEG_SLICE)],
                    deg_sp.at[pl.ds(s * DEG_SLICE, DEG_SLICE)])
    for i in range(SW // 16):
        ones_v[pl.ds(i * 16, 16)] = jnp.ones((16,), jnp.float32)
    plsc.subcore_barrier()

    tile_row0 = c * (ROWS_2D // NC) + s * A_TILE_ROWS

    def chunk(k, carry):
        r0 = tile_row0 + k * A_CHUNK_ROWS
        pltpu.sync_copy(src2d.at[pl.ds(r0, A_CHUNK_ROWS)], idx_v)
        puts = [
            pltpu.async_copy(ones_v, deg_sp.at[idx_v.at[j]], sem, add=True)
            for j in range(A_CHUNK_ROWS)
        ]
        for d in puts:
            d.wait()
        return carry

    lax.fori_loop(0, A_CHUNKS, chunk, 0)
    plsc.subcore_barrier()
    pltpu.sync_copy(deg_sp.at[pl.ds(s * DEG_SLICE, DEG_SLICE)],
                    deg_out.at[c, pl.ds(s * DEG_SLICE, DEG_SLICE)])


def _make_deg_kernel():
    mesh = plsc.VectorSubcoreMesh(core_axis_name="c", subcore_axis_name="s")
    return pl.kernel(
        _deg_body,
        out_type=jax.ShapeDtypeStruct((NC, DEG_PAD), jnp.float32),
        mesh=mesh,
        scratch_types=[
            pltpu.VMEM_SHARED((DEG_PAD,), jnp.float32),
            pltpu.VMEM((A_CHUNK_ROWS, SW), jnp.int32),
            pltpu.VMEM((SW,), jnp.float32),
            pltpu.SemaphoreType.DMA,
        ],
    )


def _agg_body(fa, fb, src2d, dst2d, zeros_hbm, out_a, out_b,
              agg_sp, idx_s0, idx_d0, rows0, idx_s1, idx_d1, rows1,
              gsem, ssem0, ssem1):
    c = lax.axis_index("c")
    s = lax.axis_index("s")
    zrows = AGG_PAD // NS
    pltpu.sync_copy(zeros_hbm.at[pl.ds(s * zrows, zrows)],
                    agg_sp.at[pl.ds(s * zrows, zrows)])
    plsc.subcore_barrier()

    tile_row0 = s * C_TILE_ROWS
    bufs = ((idx_s0, idx_d0, rows0, ssem0), (idx_s1, idx_d1, rows1, ssem1))

    def pair(k2, carry):
        for b in range(2):
            idx_s, idx_d, rows_v, ssem = bufs[b]
            k = k2 * 2 + b

            @pl.when(k2 >= 1)
            def _():
                pltpu.make_async_copy(
                    zeros_hbm.at[pl.ds(0, C_BLK)], rows_v, ssem).wait()

            r0 = tile_row0 + k * C_CHUNK_ROWS
            pltpu.sync_copy(src2d.at[pl.ds(r0, C_CHUNK_ROWS)], idx_s)
            pltpu.sync_copy(dst2d.at[pl.ds(r0, C_CHUNK_ROWS)], idx_d)
            for j in range(C_CHUNK_ROWS):
                for i in range(SW // 16):
                    sl = pl.ds(i * 16, 16)
                    sv = idx_s[j, sl]
                    idx_s[j, sl] = jnp.where(sv >= N, 0, sv)

            def gather_from(tab):
                gets = [
                    pltpu.async_copy(tab.at[idx_s.at[j]],
                                     rows_v.at[pl.ds(j * SW, SW)], gsem)
                    for j in range(C_CHUNK_ROWS)
                ]
                for d in gets:
                    d.wait()

            @pl.when(c == 0)
            def _():
                gather_from(fa)

            @pl.when(c == 1)
            def _():
                gather_from(fb)

            for j in range(C_CHUNK_ROWS):
                pltpu.async_copy(rows_v.at[pl.ds(j * SW, SW)],
                                 agg_sp.at[idx_d.at[j]], ssem, add=True)
        return carry

    lax.fori_loop(0, C_CHUNKS // 2, pair, 0)
    for b in range(2):
        _, _, rows_v, ssem = bufs[b]
        pltpu.make_async_copy(
            zeros_hbm.at[pl.ds(0, C_BLK)], rows_v, ssem).wait()
    plsc.subcore_barrier()
    big = 6256
    last = N - 15 * big

    def writeout(dst):
        @pl.when(s < NS - 1)
        def _():
            pltpu.sync_copy(agg_sp.at[pl.ds(s * big, big)],
                            dst.at[pl.ds(s * big, big)])

        @pl.when(s == NS - 1)
        def _():
            pltpu.sync_copy(agg_sp.at[pl.ds(15 * big, last)],
                            dst.at[pl.ds(15 * big, last)])

    @pl.when(c == 0)
    def _():
        writeout(out_a)

    @pl.when(c == 1)
    def _():
        writeout(out_b)


def _make_agg_kernel():
    mesh = plsc.VectorSubcoreMesh(core_axis_name="c", subcore_axis_name="s")
    return pl.kernel(
        _agg_body,
        out_type=(jax.ShapeDtypeStruct((N, FH), jnp.float32),
                  jax.ShapeDtypeStruct((N, FH), jnp.float32)),
        mesh=mesh,
        scratch_types=[
            pltpu.VMEM_SHARED((AGG_PAD, FH), jnp.float32),
            pltpu.VMEM((C_CHUNK_ROWS, SW), jnp.int32),
            pltpu.VMEM((C_CHUNK_ROWS, SW), jnp.int32),
            pltpu.VMEM((C_BLK, FH), jnp.float32),
            pltpu.VMEM((C_CHUNK_ROWS, SW), jnp.int32),
            pltpu.VMEM((C_CHUNK_ROWS, SW), jnp.int32),
            pltpu.VMEM((C_BLK, FH), jnp.float32),
            pltpu.SemaphoreType.DMA,
            pltpu.SemaphoreType.DMA,
            pltpu.SemaphoreType.DMA,
        ],
        compiler_params=pltpu.CompilerParams(use_tc_tiling_on_sc=False),
    )


def _feat_body(feat_ref, dA_ref, dB_ref, outa_ref, outb_ref):
    d = dA_ref[...] + dB_ref[...]
    norm = jnp.where(d > 0.0, lax.rsqrt(jnp.maximum(d, 1.0)), 0.0)
    f = feat_ref[...] * norm
    zero = jnp.zeros((R_BLK, 1), jnp.float32)
    one = jnp.ones((R_BLK, 1), jnp.float32)
    outa_ref[...] = f[:, :FH]
    outb_ref[...] = jnp.concatenate([f[:, FH:], zero, one], axis=1)


def _feat_kernel(features, dA, dB):
    return pl.pallas_call(
        _feat_body,
        grid=(N_BLKS,),
        in_specs=[
            pl.BlockSpec((R_BLK, 30), lambda i: (i, 0)),
            pl.BlockSpec((R_BLK, 1), lambda i: (i, 0)),
            pl.BlockSpec((R_BLK, 1), lambda i: (i, 0)),
        ],
        out_specs=[pl.BlockSpec((R_BLK, FH), lambda i: (i, 0)),
                   pl.BlockSpec((R_BLK, FH), lambda i: (i, 0))],
        out_shape=[jax.ShapeDtypeStruct((N, FH), jnp.float32),
                   jax.ShapeDtypeStruct((N, FH), jnp.float32)],
    )(features, dA, dB)


D_BLK = 1000
D_BLKS = (N // 4) // D_BLK


def _head_body(a_ref, b_ref, gid_ref, ea_ref, eb_ref, sel_ref, w1_ref,
               b1_ref, w2_ref, b2_ref, w3_ref, b3_ref, out_ref, pool_ref):
    k = pl.program_id(0)

    @pl.when(k == 0)
    def _():
        pool_ref[...] = jnp.full((G, 128), -jnp.inf, jnp.float32)

    x = (jnp.dot(a_ref[...], ea_ref[...], preferred_element_type=jnp.float32)
         + jnp.dot(b_ref[...], eb_ref[...],
                   preferred_element_type=jnp.float32))
    din = jnp.dot(x, sel_ref[...], preferred_element_type=jnp.float32)
    norm = jnp.where(din > 0.0, lax.rsqrt(jnp.maximum(din, 1.0)), 0.0)
    hr = jnp.dot(x * norm, w1_ref[...], preferred_element_type=jnp.float32)
    ids = gid_ref[...].astype(jnp.int32)
    g_lo = jnp.min(ids)
    g_hi = jnp.max(ids)

    def seg(g, carry):
        m = ids == g
        contrib = jnp.max(jnp.where(m, hr, -jnp.inf), axis=0)[None, :]
        pool_ref[pl.ds(g, 1), :] = jnp.maximum(pool_ref[pl.ds(g, 1), :],
                                               contrib)
        return carry

    lax.fori_loop(g_lo, g_hi + 1, seg, 0)

    @pl.when(k == pl.num_programs(0) - 1)
    def _():
        p = pool_ref[...]
        p = jnp.maximum(jnp.maximum(p[:, 0:32], p[:, 32:64]),
                        jnp.maximum(p[:, 64:96], p[:, 96:128]))
        p = jnp.where(jnp.isfinite(p), p + b1_ref[...], 0.0)
        z = jnp.maximum(jnp.dot(p, w2_ref[...],
                                preferred_element_type=jnp.float32)
                        + b2_ref[...], 0.0)
        o = jax.nn.sigmoid(jnp.dot(z, w3_ref[...],
                                   preferred_element_type=jnp.float32)
                           + b3_ref[...])
        out_ref[...] = o


def _head_kernel(a4, b4, gid128, ea, eb, sel, w1bd, b1p, w2p, b2p, w3p, b3p):
    full = lambda a: pl.BlockSpec(a.shape, lambda i: tuple(0 for _ in a.shape))
    return pl.pallas_call(
        _head_body,
        grid=(D_BLKS,),
        in_specs=[
            pl.BlockSpec((D_BLK, 64), lambda i: (i, 0)),
            pl.BlockSpec((D_BLK, 64), lambda i: (i, 0)),
            pl.BlockSpec((D_BLK, 128), lambda i: (i, 0)),
            full(ea), full(eb), full(sel), full(w1bd), full(b1p),
            full(w2p), full(b2p), full(w3p), full(b3p),
        ],
        out_specs=pl.BlockSpec((G, 8), lambda i: (0, 0)),
        out_shape=jax.ShapeDtypeStruct((G, 8), jnp.float32),
        scratch_shapes=[pltpu.VMEM((G, 128), jnp.float32)],
    )(a4, b4, gid128, ea, eb, sel, w1bd, b1p, w2p, b2p, w3p, b3p)


@jax.jit
def kernel(features, edge_index, graph_ids, W1, b1, W2, b2, W3, b3):
    src = edge_index[0].astype(jnp.int32)
    dst = edge_index[1].astype(jnp.int32)
    pad = E_PAD - E
    src2d = jnp.concatenate(
        [src, jnp.full((pad,), SRC_PAD, jnp.int32)]).reshape(ROWS_2D, SW)
    dst2d = jnp.concatenate(
        [dst, jnp.full((pad,), DST_PAD, jnp.int32)]).reshape(ROWS_2D, SW)

    degs = _make_deg_kernel()(src2d, jnp.zeros((DEG_PAD,), jnp.float32))
    dA = degs[0, :N].reshape(N, 1)
    dB = degs[1, :N].reshape(N, 1)

    feat_a, feat_b = _feat_kernel(features, dA, dB)

    agg_a, agg_b = _make_agg_kernel()(
        feat_a, feat_b, src2d, dst2d,
        jnp.zeros((AGG_PAD, FH), jnp.float32))

    a4 = agg_a.reshape(N // 4, 64)
    b4 = agg_b.reshape(N // 4, 64)
    gid128 = jnp.repeat(graph_ids.astype(jnp.int8), 32).reshape(N // 4, 128)

    r64 = jnp.arange(64)
    ea = jnp.zeros((64, 128), jnp.float32).at[
        r64, 32 * (r64 // 16) + r64 % 16].set(1.0)
    eb = jnp.zeros((64, 128), jnp.float32).at[
        r64, 32 * (r64 // 16) + 16 + r64 % 16].set(1.0)
    basis = jnp.zeros((32, 32), jnp.float32).at[31, :].set(1.0)
    sel = jnp.kron(jnp.eye(4, dtype=jnp.float32), basis)
    w1p = jnp.pad(W1, ((0, 2), (0, 2)))
    w1bd = jnp.kron(jnp.eye(4, dtype=jnp.float32), w1p)
    b1p = jnp.pad(b1, (0, 2)).reshape(1, 32)
    w2p = jnp.pad(W2, ((0, 2), (0, 6)))
    b2p = jnp.pad(b2, (0, 6)).reshape(1, 16)
    w3p = jnp.pad(W3, ((0, 6), (0, 4)))
    b3p = jnp.pad(b3, (0, 4)).reshape(1, 8)

    out8 = _head_kernel(a4, b4, gid128, ea, eb, sel, w1bd, b1p, w2p, b2p,
                        w3p, b3p)
    return out8[:, :4]


# speedup vs baseline: 1.3404x; 1.3404x over previous
import jax
import jax.numpy as jnp
from jax import lax
from jax.experimental import pallas as pl
from jax.experimental.pallas import tpu as pltpu
from jax.experimental.pallas import tpu_sc as plsc

N = 100000
E = 1600000
G = 128
FH = 16

NC, NS = 2, 16
CHUNK = 1024

A_TILE = E // (NC * NS)
A_FULL = A_TILE // CHUNK
A_REM = A_TILE - A_FULL * CHUNK
DEG_PAD = 100352
DEG_SLICE = DEG_PAD // NS

C_TILE = E // NS
C_FULL = C_TILE // CHUNK
C_REM = C_TILE - C_FULL * CHUNK
AGG_PAD = 100008
DUMMY = N
BIG = 6256
LAST = N - (NS - 1) * BIG
ZLAST = AGG_PAD - (NS - 1) * BIG

R_BLK = 2000
N_BLKS = N // R_BLK


def _deg_body(src1d, zeros_hbm, deg_out, deg_sp, idx_v, ones_v, sem):
    c = lax.axis_index("c")
    s = lax.axis_index("s")
    pltpu.sync_copy(zeros_hbm.at[pl.ds(s * DEG_SLICE, DEG_SLICE)],
                    deg_sp.at[pl.ds(s * DEG_SLICE, DEG_SLICE)])
    for i in range(CHUNK // 16):
        ones_v[pl.ds(i * 16, 16)] = jnp.ones((16,), jnp.float32)
    plsc.subcore_barrier()

    e0 = (c * NS + s) * A_TILE

    def chunk(k, carry):
        base = e0 + k * CHUNK
        pltpu.sync_copy(src1d.at[pl.ds(base, CHUNK)], idx_v)
        pltpu.sync_copy(ones_v, deg_sp.at[idx_v], add=True)
        return carry

    lax.fori_loop(0, A_FULL, chunk, 0)
    base = e0 + A_FULL * CHUNK
    pltpu.sync_copy(src1d.at[pl.ds(base, A_REM)], idx_v.at[pl.ds(0, A_REM)])
    for i in range(A_REM // 16, CHUNK // 16):
        idx_v[pl.ds(i * 16, 16)] = jnp.full((16,), N, jnp.int32)
    pltpu.sync_copy(ones_v, deg_sp.at[idx_v], add=True)
    plsc.subcore_barrier()
    pltpu.sync_copy(deg_sp.at[pl.ds(s * DEG_SLICE, DEG_SLICE)],
                    deg_out.at[c, pl.ds(s * DEG_SLICE, DEG_SLICE)])


def _make_deg_kernel():
    mesh = plsc.VectorSubcoreMesh(core_axis_name="c", subcore_axis_name="s")
    return pl.kernel(
        _deg_body,
        out_type=jax.ShapeDtypeStruct((NC, DEG_PAD), jnp.float32),
        mesh=mesh,
        scratch_types=[
            pltpu.VMEM_SHARED((DEG_PAD,), jnp.float32),
            pltpu.VMEM((CHUNK,), jnp.int32),
            pltpu.VMEM((CHUNK,), jnp.float32),
            pltpu.SemaphoreType.DMA,
        ],
        compiler_params=pltpu.CompilerParams(use_tc_tiling_on_sc=False),
    )


def _agg_body(fa, fb, src1d, dst1d, zeros_hbm, out_a, out_b,
              agg_sp, idx_s, idx_d, rows_v, gsem, ssem):
    c = lax.axis_index("c")
    s = lax.axis_index("s")

    @pl.when(s < NS - 1)
    def _():
        pltpu.sync_copy(zeros_hbm.at[pl.ds(s * BIG, BIG)],
                        agg_sp.at[pl.ds(s * BIG, BIG)])

    @pl.when(s == NS - 1)
    def _():
        pltpu.sync_copy(zeros_hbm.at[pl.ds((NS - 1) * BIG, ZLAST)],
                        agg_sp.at[pl.ds((NS - 1) * BIG, ZLAST)])

    plsc.subcore_barrier()

    e0 = s * C_TILE

    def do_chunk():
        @pl.when(c == 0)
        def _():
            pltpu.async_copy(fa.at[idx_s], rows_v, gsem).wait()

        @pl.when(c == 1)
        def _():
            pltpu.async_copy(fb.at[idx_s], rows_v, gsem).wait()

        pltpu.sync_copy(rows_v, agg_sp.at[idx_d], add=True)

    def chunk(k, carry):
        base = e0 + k * CHUNK
        pltpu.sync_copy(src1d.at[pl.ds(base, CHUNK)], idx_s)
        pltpu.sync_copy(dst1d.at[pl.ds(base, CHUNK)], idx_d)
        do_chunk()
        return carry

    lax.fori_loop(0, C_FULL, chunk, 0)
    base = e0 + C_FULL * CHUNK
    pltpu.sync_copy(src1d.at[pl.ds(base, C_REM)], idx_s.at[pl.ds(0, C_REM)])
    pltpu.sync_copy(dst1d.at[pl.ds(base, C_REM)], idx_d.at[pl.ds(0, C_REM)])
    for i in range(C_REM // 16, CHUNK // 16):
        idx_s[pl.ds(i * 16, 16)] = jnp.zeros((16,), jnp.int32)
        idx_d[pl.ds(i * 16, 16)] = jnp.full((16,), DUMMY, jnp.int32)
    do_chunk()
    plsc.subcore_barrier()

    def writeout(dst):
        @pl.when(s < NS - 1)
        def _():
            pltpu.sync_copy(agg_sp.at[pl.ds(s * BIG, BIG)],
                            dst.at[pl.ds(s * BIG, BIG)])

        @pl.when(s == NS - 1)
        def _():
            pltpu.sync_copy(agg_sp.at[pl.ds((NS - 1) * BIG, LAST)],
                            dst.at[pl.ds((NS - 1) * BIG, LAST)])

    @pl.when(c == 0)
    def _():
        writeout(out_a)

    @pl.when(c == 1)
    def _():
        writeout(out_b)


def _make_agg_kernel():
    mesh = plsc.VectorSubcoreMesh(core_axis_name="c", subcore_axis_name="s")
    return pl.kernel(
        _agg_body,
        out_type=(jax.ShapeDtypeStruct((N, FH), jnp.float32),
                  jax.ShapeDtypeStruct((N, FH), jnp.float32)),
        mesh=mesh,
        scratch_types=[
            pltpu.VMEM_SHARED((AGG_PAD, FH), jnp.float32),
            pltpu.VMEM((CHUNK,), jnp.int32),
            pltpu.VMEM((CHUNK,), jnp.int32),
            pltpu.VMEM((CHUNK, FH), jnp.float32),
            pltpu.SemaphoreType.DMA,
            pltpu.SemaphoreType.DMA,
        ],
        compiler_params=pltpu.CompilerParams(use_tc_tiling_on_sc=False),
    )


def _feat_body(feat_ref, dA_ref, dB_ref, outa_ref, outb_ref):
    d = dA_ref[...] + dB_ref[...]
    norm = jnp.where(d > 0.0, lax.rsqrt(jnp.maximum(d, 1.0)), 0.0)
    f = feat_ref[...] * norm
    zero = jnp.zeros((R_BLK, 1), jnp.float32)
    one = jnp.ones((R_BLK, 1), jnp.float32)
    outa_ref[...] = f[:, :FH]
    outb_ref[...] = jnp.concatenate([f[:, FH:], zero, one], axis=1)


def _feat_kernel(features, dA, dB):
    return pl.pallas_call(
        _feat_body,
        grid=(N_BLKS,),
        in_specs=[
            pl.BlockSpec((R_BLK, 30), lambda i: (i, 0)),
            pl.BlockSpec((R_BLK, 1), lambda i: (i, 0)),
            pl.BlockSpec((R_BLK, 1), lambda i: (i, 0)),
        ],
        out_specs=[pl.BlockSpec((R_BLK, FH), lambda i: (i, 0)),
                   pl.BlockSpec((R_BLK, FH), lambda i: (i, 0))],
        out_shape=[jax.ShapeDtypeStruct((N, FH), jnp.float32),
                   jax.ShapeDtypeStruct((N, FH), jnp.float32)],
    )(features, dA, dB)


D_BLK = 1000
D_BLKS = (N // 4) // D_BLK


def _head_body(a_ref, b_ref, gid_ref, ea_ref, eb_ref, sel_ref, w1_ref,
               b1_ref, w2_ref, b2_ref, w3_ref, b3_ref, out_ref, pool_ref):
    k = pl.program_id(0)

    @pl.when(k == 0)
    def _():
        pool_ref[...] = jnp.full((G, 128), -jnp.inf, jnp.float32)

    x = (jnp.dot(a_ref[...], ea_ref[...], preferred_element_type=jnp.float32)
         + jnp.dot(b_ref[...], eb_ref[...],
                   preferred_element_type=jnp.float32))
    din = jnp.dot(x, sel_ref[...], preferred_element_type=jnp.float32)
    norm = jnp.where(din > 0.0, lax.rsqrt(jnp.maximum(din, 1.0)), 0.0)
    hr = jnp.dot(x * norm, w1_ref[...], preferred_element_type=jnp.float32)
    ids = gid_ref[...].astype(jnp.int32)
    g_lo = jnp.min(ids)
    g_hi = jnp.max(ids)

    def seg(g, carry):
        m = ids == g
        contrib = jnp.max(jnp.where(m, hr, -jnp.inf), axis=0)[None, :]
        pool_ref[pl.ds(g, 1), :] = jnp.maximum(pool_ref[pl.ds(g, 1), :],
                                               contrib)
        return carry

    lax.fori_loop(g_lo, g_hi + 1, seg, 0)

    @pl.when(k == pl.num_programs(0) - 1)
    def _():
        p = pool_ref[...]
        p = jnp.maximum(jnp.maximum(p[:, 0:32], p[:, 32:64]),
                        jnp.maximum(p[:, 64:96], p[:, 96:128]))
        p = jnp.where(jnp.isfinite(p), p + b1_ref[...], 0.0)
        z = jnp.maximum(jnp.dot(p, w2_ref[...],
                                preferred_element_type=jnp.float32)
                        + b2_ref[...], 0.0)
        o = jax.nn.sigmoid(jnp.dot(z, w3_ref[...],
                                   preferred_element_type=jnp.float32)
                           + b3_ref[...])
        out_ref[...] = o


def _head_kernel(a4, b4, gid128, ea, eb, sel, w1bd, b1p, w2p, b2p, w3p, b3p):
    full = lambda a: pl.BlockSpec(a.shape, lambda i: tuple(0 for _ in a.shape))
    return pl.pallas_call(
        _head_body,
        grid=(D_BLKS,),
        in_specs=[
            pl.BlockSpec((D_BLK, 64), lambda i: (i, 0)),
            pl.BlockSpec((D_BLK, 64), lambda i: (i, 0)),
            pl.BlockSpec((D_BLK, 128), lambda i: (i, 0)),
            full(ea), full(eb), full(sel), full(w1bd), full(b1p),
            full(w2p), full(b2p), full(w3p), full(b3p),
        ],
        out_specs=pl.BlockSpec((G, 8), lambda i: (0, 0)),
        out_shape=jax.ShapeDtypeStruct((G, 8), jnp.float32),
        scratch_shapes=[pltpu.VMEM((G, 128), jnp.float32)],
    )(a4, b4, gid128, ea, eb, sel, w1bd, b1p, w2p, b2p, w3p, b3p)


@jax.jit
def kernel(features, edge_index, graph_ids, W1, b1, W2, b2, W3, b3):
    src1d = edge_index[0].astype(jnp.int32)
    dst1d = edge_index[1].astype(jnp.int32)

    degs = _make_deg_kernel()(src1d, jnp.zeros((DEG_PAD,), jnp.float32))
    dA = degs[0, :N].reshape(N, 1)
    dB = degs[1, :N].reshape(N, 1)

    feat_a, feat_b = _feat_kernel(features, dA, dB)

    agg_a, agg_b = _make_agg_kernel()(
        feat_a, feat_b, src1d, dst1d, jnp.zeros((AGG_PAD, FH), jnp.float32))

    a4 = agg_a.reshape(N // 4, 64)
    b4 = agg_b.reshape(N // 4, 64)
    gid128 = jnp.repeat(graph_ids.astype(jnp.int8), 32).reshape(N // 4, 128)

    r64 = jnp.arange(64)
    ea = jnp.zeros((64, 128), jnp.float32).at[
        r64, 32 * (r64 // 16) + r64 % 16].set(1.0)
    eb = jnp.zeros((64, 128), jnp.float32).at[
        r64, 32 * (r64 // 16) + 16 + r64 % 16].set(1.0)
    basis = jnp.zeros((32, 32), jnp.float32).at[31, :].set(1.0)
    sel = jnp.kron(jnp.eye(4, dtype=jnp.float32), basis)
    w1p = jnp.pad(W1, ((0, 2), (0, 2)))
    w1bd = jnp.kron(jnp.eye(4, dtype=jnp.float32), w1p)
    b1p = jnp.pad(b1, (0, 2)).reshape(1, 32)
    w2p = jnp.pad(W2, ((0, 2), (0, 6)))
    b2p = jnp.pad(b2, (0, 6)).reshape(1, 16)
    w3p = jnp.pad(W3, ((0, 6), (0, 4)))
    b3p = jnp.pad(b3, (0, 4)).reshape(1, 8)

    out8 = _head_kernel(a4, b4, gid128, ea, eb, sel, w1bd, b1p, w2p, b2p,
                        w3p, b3p)
    return out8[:, :4]
